# baseline (speedup 1.0000x reference)
import functools

import jax
import jax.numpy as jnp
from jax import lax
from jax.experimental import pallas as pl
from jax.experimental.pallas import tpu as pltpu
from jax.experimental.pallas import tpu_sc as plsc

NUM_ROWS = 100000
D = 64
DP = 128
BATCH = 4096
HIST = 200
NUM_WORKERS = 32
BSTRIPE = BATCH // NUM_WORKERS
L = 16
NGRP = BSTRIPE // L


def _sc_gather_t(idx_t, table_pad):
    mesh = plsc.VectorSubcoreMesh(core_axis_name="c", subcore_axis_name="s")

    @functools.partial(
        pl.kernel,
        mesh=mesh,
        out_type=jax.ShapeDtypeStruct((HIST, D, BATCH), jnp.float32),
        compiler_params=pltpu.CompilerParams(
            use_tc_tiling_on_sc=True, needs_layout_passes=False),
        scratch_types=(
            [pltpu.VMEM((HIST, BSTRIPE), jnp.int32)]
            + [pltpu.VMEM((BSTRIPE, DP), jnp.float32) for _ in range(2)]
            + [pltpu.VMEM((D, BSTRIPE), jnp.float32) for _ in range(2)]
            + [pltpu.SemaphoreType.DMA for _ in range(4)]
        ),
    )
    def k(idx_ref, table_ref, out_ref, idx_v, rows_a, rows_b, trans_a,
          trans_b, gsem_a, gsem_b, ssem_a, ssem_b):
        wid = lax.axis_index("s") * 2 + lax.axis_index("c")
        b0 = wid * BSTRIPE
        iota = lax.iota(jnp.int32, L)
        rowv = [bg * L + iota for bg in range(NGRP)]

        pltpu.sync_copy(idx_ref.at[:, pl.ds(b0, BSTRIPE)], idx_v)

        def gather_start(rows, sem, h):
            pltpu.async_copy(table_ref.at[idx_v.at[h]], rows, sem)

        def gather_wait(rows, sem):
            pltpu.make_async_copy(table_ref.at[idx_v.at[0]], rows, sem).wait()

        def store_start(trans, sem, h):
            pltpu.async_copy(trans, out_ref.at[h, :, pl.ds(b0, BSTRIPE)], sem)

        def store_wait(trans, sem):
            pltpu.make_async_copy(trans, out_ref.at[0, :, pl.ds(b0, BSTRIPE)],
                                  sem).wait()

        def transpose(rows, trans):
            @functools.partial(plsc.parallel_loop, 0, D, unroll=8)
            def _(d):
                col = jnp.full((L,), d, jnp.int32)
                for bg in range(NGRP):
                    trans[d, pl.ds(bg * L, L)] = plsc.load_gather(
                        rows, [rowv[bg], col])

        gather_start(rows_a, gsem_a, 0)
        gather_start(rows_b, gsem_b, 1)

        def round_body(r, carry):
            for h_off, rows, trans, gsem, ssem in (
                    (0, rows_a, trans_a, gsem_a, ssem_a),
                    (1, rows_b, trans_b, gsem_b, ssem_b)):
                h = 2 * r + h_off
                gather_wait(rows, gsem)

                @pl.when(r > 0)
                def _():
                    store_wait(trans, ssem)

                transpose(rows, trans)
                store_start(trans, ssem, h)

                @pl.when(r < HIST // 2 - 1)
                def _():
                    gather_start(rows, gsem, h + 2)

            return carry

        lax.fori_loop(0, HIST // 2, round_body, 0)
        store_wait(trans_a, ssem_a)
        store_wait(trans_b, ssem_b)

    return k(idx_t, table_pad)


def kernel(indices, atom_embedding):
    idx_t = indices.astype(jnp.int32).T
    table_pad = jnp.pad(atom_embedding, ((0, 0), (0, DP - D)))
    out_t = _sc_gather_t(idx_t, table_pad)
    return jnp.transpose(out_t, (2, 0, 1))


# speedup vs baseline: 1.0287x; 1.0287x over previous
import functools

import jax
import jax.numpy as jnp
from jax import lax
from jax.experimental import pallas as pl
from jax.experimental.pallas import tpu as pltpu
from jax.experimental.pallas import tpu_sc as plsc

NUM_ROWS = 100000
D = 64
DP = 128
BATCH = 4096
HIST = 200
NUM_WORKERS = 32
BSTRIPE = BATCH // NUM_WORKERS
L = 16
NGRP = BSTRIPE // L


def _sc_gather_t(idx_t, table_pad):
    mesh = plsc.VectorSubcoreMesh(core_axis_name="c", subcore_axis_name="s")

    @functools.partial(
        pl.kernel,
        mesh=mesh,
        out_type=jax.ShapeDtypeStruct((HIST, D, BATCH), jnp.float32),
        compiler_params=pltpu.CompilerParams(
            use_tc_tiling_on_sc=True, needs_layout_passes=False),
        scratch_types=(
            [pltpu.VMEM((HIST, BSTRIPE), jnp.int32)]
            + [pltpu.VMEM((BSTRIPE, DP), jnp.float32) for _ in range(4)]
            + [pltpu.VMEM((D, BSTRIPE), jnp.float32) for _ in range(2)]
            + [pltpu.SemaphoreType.DMA for _ in range(6)]
        ),
    )
    def k(idx_ref, table_ref, out_ref, idx_v, *scratch):
        rows = scratch[0:4]
        trans = scratch[4:6]
        gsem = scratch[6:10]
        ssem = scratch[10:12]
        wid = lax.axis_index("s") * 2 + lax.axis_index("c")
        b0 = wid * BSTRIPE
        iota = lax.iota(jnp.int32, L)
        rowv = [bg * L + iota for bg in range(NGRP)]

        pltpu.sync_copy(idx_ref.at[:, pl.ds(b0, BSTRIPE)], idx_v)

        def gather_start(rows, sem, h):
            pltpu.async_copy(table_ref.at[idx_v.at[h]], rows, sem)

        def gather_wait(rows, sem):
            pltpu.make_async_copy(table_ref.at[idx_v.at[0]], rows, sem).wait()

        def store_start(trans, sem, h):
            pltpu.async_copy(trans, out_ref.at[h, :, pl.ds(b0, BSTRIPE)], sem)

        def store_wait(trans, sem):
            pltpu.make_async_copy(trans, out_ref.at[0, :, pl.ds(b0, BSTRIPE)],
                                  sem).wait()

        def transpose(rows_v, trans_v):
            @functools.partial(plsc.parallel_loop, 0, D, unroll=4)
            def _(d):
                col = jnp.full((L,), d, jnp.int32)
                for bg in range(NGRP):
                    trans_v[d, pl.ds(bg * L, L)] = plsc.load_gather(
                        rows_v, [rowv[bg], col])

        NB = 4
        for o in range(NB):
            gather_start(rows[o], gsem[o], o)

        def round_body(r, carry):
            for o in range(NB):
                h = NB * r + o
                gather_wait(rows[o], gsem[o])

                if o >= 2:
                    store_wait(trans[o % 2], ssem[o % 2])
                else:
                    @pl.when(r > 0)
                    def _():
                        store_wait(trans[o % 2], ssem[o % 2])

                transpose(rows[o], trans[o % 2])
                store_start(trans[o % 2], ssem[o % 2], h)

                @pl.when(r < HIST // NB - 1)
                def _():
                    gather_start(rows[o], gsem[o], h + NB)

            return carry

        lax.fori_loop(0, HIST // NB, round_body, 0)
        store_wait(trans[0], ssem[0])
        store_wait(trans[1], ssem[1])

    return k(idx_t, table_pad)


def kernel(indices, atom_embedding):
    idx_t = indices.astype(jnp.int32).T
    table_pad = jnp.pad(atom_embedding, ((0, 0), (0, DP - D)))
    out_t = _sc_gather_t(idx_t, table_pad)
    return jnp.transpose(out_t, (2, 0, 1))
